# Initial kernel scaffold; baseline (speedup 1.0000x reference)
import jax
import jax.numpy as jnp
from jax.experimental import pallas as pl


def kernel(superposition_weights, W):
    raise NotImplementedError("write your pallas kernel here")


# speedup vs baseline: 1.4085x; 1.4085x over previous
import jax
import jax.numpy as jnp
from jax.experimental import pallas as pl


def _body(w_ref, x_ref, o_ref):
    o_ref[...] = jax.lax.dot_general(
        w_ref[...], x_ref[...],
        dimension_numbers=(((0,), (0,)), ((), ())),
        preferred_element_type=jnp.float32,
    )


def kernel(superposition_weights, W):
    E, B = superposition_weights.shape
    _, d1, d2 = W.shape
    N = d1 * d2
    Wf = W.reshape(E, N)
    CH = 4096
    out = pl.pallas_call(
        _body,
        grid=(N // CH,),
        in_specs=[
            pl.BlockSpec((E, B), lambda i: (0, 0)),
            pl.BlockSpec((E, CH), lambda i: (0, i)),
        ],
        out_specs=pl.BlockSpec((B, CH), lambda i: (0, i)),
        out_shape=jax.ShapeDtypeStruct((B, N), jnp.float32),
    )(superposition_weights, Wf)
    return out.reshape(B, d1, d2)


# speedup vs baseline: 4.5269x; 3.2140x over previous
import jax
import jax.numpy as jnp
from jax.experimental import pallas as pl

_RBLK = 16


def _body(w_ref, x_ref, o_ref):
    w = w_ref[...]
    for r in range(_RBLK):
        o_ref[:, r, :] = jax.lax.dot_general(
            w, x_ref[:, r, :],
            dimension_numbers=(((0,), (0,)), ((), ())),
            preferred_element_type=jnp.float32,
        )


def kernel(superposition_weights, W):
    E, B = superposition_weights.shape
    _, d1, d2 = W.shape
    out = pl.pallas_call(
        _body,
        grid=(d1 // _RBLK,),
        in_specs=[
            pl.BlockSpec((E, B), lambda i: (0, 0)),
            pl.BlockSpec((E, _RBLK, d2), lambda i: (0, i, 0)),
        ],
        out_specs=pl.BlockSpec((B, _RBLK, d2), lambda i: (0, i, 0)),
        out_shape=jax.ShapeDtypeStruct((B, d1, d2), jnp.float32),
    )(superposition_weights, W)
    return out
